# Initial kernel scaffold; baseline (speedup 1.0000x reference)
import jax
import jax.numpy as jnp
from jax.experimental import pallas as pl


def kernel(edge_index, edge_attr, x, W, b):
    raise NotImplementedError("write your pallas kernel here")


# speedup vs baseline: 2.2966x; 2.2966x over previous
import functools

import jax
import jax.numpy as jnp
from jax import lax
from jax.experimental import pallas as pl
from jax.experimental.pallas import tpu as pltpu
from jax.experimental.pallas import tpu_sc as plsc

_LANES = 16


def _proj_tc(edge_attr, Wt, b2d):
    E, R = edge_attr.shape
    H = Wt.shape[1]
    BE = 2000
    assert E % BE == 0

    def body(ea_ref, wt_ref, b_ref, out_ref):
        out_ref[...] = (
            jnp.dot(ea_ref[...], wt_ref[...], preferred_element_type=jnp.float32)
            + b_ref[...]
        )

    return pl.pallas_call(
        body,
        grid=(E // BE,),
        in_specs=[
            pl.BlockSpec((BE, R), lambda i: (i, 0)),
            pl.BlockSpec((R, H), lambda i: (0, 0)),
            pl.BlockSpec((1, H), lambda i: (0, 0)),
        ],
        out_specs=pl.BlockSpec((BE, H), lambda i: (i, 0)),
        out_shape=jax.ShapeDtypeStruct((E, H), jnp.float32),
    )(edge_attr, Wt, b2d)


def _sc_combine(src, dst, proj, x):
    E = src.shape[0]
    V, H = x.shape
    info = plsc.get_sparse_core_info()
    NC, NS = info.num_cores, info.num_subcores
    NW = NC * NS
    assert E % NW == 0
    epw = E // NW
    B = 200
    assert epw % B == 0
    nblk = epw // B
    HC = H // _LANES

    mesh = plsc.VectorSubcoreMesh(core_axis_name="c", subcore_axis_name="s")

    @functools.partial(
        pl.kernel,
        mesh=mesh,
        out_type=jax.ShapeDtypeStruct((E, H), jnp.float32),
        scratch_types=[
            pltpu.VMEM((B,), jnp.int32),
            pltpu.VMEM((B,), jnp.int32),
            pltpu.VMEM((B, H), jnp.float32),
            pltpu.VMEM((B, H), jnp.float32),
            pltpu.VMEM((B, H), jnp.float32),
            pltpu.SemaphoreType.DMA,
            pltpu.SemaphoreType.DMA,
        ],
    )
    def k(src_hbm, dst_hbm, proj_hbm, x_hbm, out_hbm,
          idx_s, idx_d, xi_v, xj_v, pr_v, sem_i, sem_j):
        wid = lax.axis_index("s") * NC + lax.axis_index("c")
        wbase = wid * epw

        def block(g, carry):
            base = wbase + g * B
            pltpu.sync_copy(src_hbm.at[pl.ds(base, B)], idx_s)
            pltpu.sync_copy(dst_hbm.at[pl.ds(base, B)], idx_d)
            cp_i = pltpu.async_copy(x_hbm.at[idx_s], xi_v, sem_i)
            cp_j = pltpu.async_copy(x_hbm.at[idx_d], xj_v, sem_j)
            pltpu.sync_copy(proj_hbm.at[pl.ds(base, B), :], pr_v)
            cp_i.wait()
            cp_j.wait()

            def edge(e, c2):
                for c in range(HC):
                    s = pl.ds(c * _LANES, _LANES)
                    xi_v[e, s] = (xi_v[e, s] + xj_v[e, s]) * pr_v[e, s]
                return c2

            lax.fori_loop(0, B, edge, 0)
            pltpu.sync_copy(xi_v, out_hbm.at[pl.ds(base, B), :])
            return carry

        lax.fori_loop(0, nblk, block, 0)

    return k(src, dst, proj, x)


def kernel(edge_index, edge_attr, x, W, b):
    src = edge_index[0].astype(jnp.int32)
    dst = edge_index[1].astype(jnp.int32)
    H = W.shape[0]
    proj = _proj_tc(edge_attr, W.T, b.reshape(1, H))
    return _sc_combine(src, dst, proj, x)


# speedup vs baseline: 3.0430x; 1.3250x over previous
import functools

import jax
import jax.numpy as jnp
from jax import lax
from jax.experimental import pallas as pl
from jax.experimental.pallas import tpu as pltpu
from jax.experimental.pallas import tpu_sc as plsc

_LANES = 16


def _proj_tc(edge_attr, Wt, b2d):
    E, R = edge_attr.shape
    H = Wt.shape[1]
    BE = 2000
    assert E % BE == 0

    def body(ea_ref, wt_ref, b_ref, out_ref):
        out_ref[...] = (
            jnp.dot(ea_ref[...], wt_ref[...], preferred_element_type=jnp.float32)
            + b_ref[...]
        )

    return pl.pallas_call(
        body,
        grid=(E // BE,),
        in_specs=[
            pl.BlockSpec((BE, R), lambda i: (i, 0)),
            pl.BlockSpec((R, H), lambda i: (0, 0)),
            pl.BlockSpec((1, H), lambda i: (0, 0)),
        ],
        out_specs=pl.BlockSpec((BE, H), lambda i: (i, 0)),
        out_shape=jax.ShapeDtypeStruct((E, H), jnp.float32),
    )(edge_attr, Wt, b2d)


def _sc_combine(src, dst, proj, x):
    E = src.shape[0]
    V, H = x.shape
    info = plsc.get_sparse_core_info()
    NC, NS = info.num_cores, info.num_subcores
    NW = NC * NS
    assert E % NW == 0
    epw = E // NW
    B = 80
    assert epw % B == 0
    nblk = epw // B
    assert nblk % 2 == 1
    HC = H // _LANES

    mesh = plsc.VectorSubcoreMesh(core_axis_name="c", subcore_axis_name="s")

    @functools.partial(
        pl.kernel,
        mesh=mesh,
        out_type=jax.ShapeDtypeStruct((E, H), jnp.float32),
        scratch_types=(
            [pltpu.VMEM((B,), jnp.int32) for _ in range(4)]
            + [pltpu.VMEM((B, H), jnp.float32) for _ in range(8)]
            + [pltpu.SemaphoreType.DMA for _ in range(12)]
        ),
    )
    def k(src_hbm, dst_hbm, proj_hbm, x_hbm, out_hbm,
          is0, is1, id0, id1, xi0, xi1, xj0, xj1, pr0, pr1, ob0, ob1,
          sis0, sis1, sid0, sid1, sgi0, sgi1, sgj0, sgj1, spr0, spr1,
          sou0, sou1):
        idx_s, idx_d = (is0, is1), (id0, id1)
        xi, xj, pr, ob = (xi0, xi1), (xj0, xj1), (pr0, pr1), (ob0, ob1)
        sis, sid = (sis0, sis1), (sid0, sid1)
        sgi, sgj, spr, sou = (sgi0, sgi1), (sgj0, sgj1), (spr0, spr1), (sou0, sou1)

        wid = lax.axis_index("s") * NC + lax.axis_index("c")
        wbase = wid * epw

        def issue_idx(g, p):
            base = wbase + g * B
            pltpu.async_copy(src_hbm.at[pl.ds(base, B)], idx_s[p], sis[p])
            pltpu.async_copy(dst_hbm.at[pl.ds(base, B)], idx_d[p], sid[p])

        def wait_idx(p):
            pltpu.make_async_copy(src_hbm.at[pl.ds(0, B)], idx_s[p], sis[p]).wait()
            pltpu.make_async_copy(dst_hbm.at[pl.ds(0, B)], idx_d[p], sid[p]).wait()

        def issue_fetch(g, p):
            base = wbase + g * B
            pltpu.async_copy(x_hbm.at[idx_s[p]], xi[p], sgi[p])
            pltpu.async_copy(x_hbm.at[idx_d[p]], xj[p], sgj[p])
            pltpu.async_copy(proj_hbm.at[pl.ds(base, B), :], pr[p], spr[p])

        def wait_fetch(p):
            pltpu.make_async_copy(x_hbm.at[idx_s[p]], xi[p], sgi[p]).wait()
            pltpu.make_async_copy(x_hbm.at[idx_d[p]], xj[p], sgj[p]).wait()
            pltpu.make_async_copy(proj_hbm.at[pl.ds(0, B), :], pr[p], spr[p]).wait()

        def issue_out(g, p):
            base = wbase + g * B
            pltpu.async_copy(ob[p], out_hbm.at[pl.ds(base, B), :], sou[p])

        def wait_out(p):
            pltpu.make_async_copy(ob[p], out_hbm.at[pl.ds(0, B), :], sou[p]).wait()

        def combine(p):
            xi_p, xj_p, pr_p, ob_p = xi[p], xj[p], pr[p], ob[p]

            def edge(e, c2):
                for c in range(HC):
                    s = pl.ds(c * _LANES, _LANES)
                    ob_p[e, s] = (xi_p[e, s] + xj_p[e, s]) * pr_p[e, s]
                return c2

            lax.fori_loop(0, B, edge, 0)

        def step(g, p):
            wait_fetch(p)
            wait_idx(1 - p)
            issue_fetch(g + 1, 1 - p)
            pl.when(g + 2 <= nblk - 1)(lambda: issue_idx(g + 2, p))
            pl.when(g >= 2)(lambda: wait_out(p))
            combine(p)
            issue_out(g, p)

        issue_idx(0, 0)
        wait_idx(0)
        issue_fetch(0, 0)
        issue_idx(1, 1)

        def pair(i, carry):
            step(2 * i, 0)
            step(2 * i + 1, 1)
            return carry

        lax.fori_loop(0, (nblk - 1) // 2, pair, 0)

        g_last = nblk - 1
        wait_fetch(0)
        wait_out(0)
        combine(0)
        issue_out(g_last, 0)
        wait_out(1)
        wait_out(0)

    return k(src, dst, proj, x)


def kernel(edge_index, edge_attr, x, W, b):
    src = edge_index[0].astype(jnp.int32)
    dst = edge_index[1].astype(jnp.int32)
    H = W.shape[0]
    proj = _proj_tc(edge_attr, W.T, b.reshape(1, H))
    return _sc_combine(src, dst, proj, x)
